# baseline (speedup 1.0000x reference)
import functools

import jax
import jax.numpy as jnp
from jax import lax
from jax.experimental import pallas as pl
from jax.experimental.pallas import tpu as pltpu
from jax.experimental.pallas import tpu_sc as plsc

BATCH = 4096
SEQ = 2050
NUM_WORKERS = 32
ROWS_PER_WORKER = BATCH // NUM_WORKERS
CHUNK_ROWS = 8
CHUNKS = ROWS_PER_WORKER // CHUNK_ROWS
VECS_PER_ROW = 2048 // 16


def _emit(in_v, p_v, r_v, rowv, jv):
    w = plsc.load_gather(in_v, [rowv, jv])
    m = lax.rem(jv - 1, jnp.full((16,), 4, jnp.int32))
    tok = plsc.load_gather(in_v, [rowv, jv - m])
    inb = (jv >= 1) & (jv <= 2048)
    pos = jnp.where(inb & (w != 0) & (w != 1), m + 1, 0)
    rbt = jnp.where(inb & (m != 0) & (tok != 1), tok, 0)
    plsc.store_scatter(p_v, [rowv, jv], pos)
    plsc.store_scatter(r_v, [rowv, jv], rbt)


def _make_kernel():
    mesh = plsc.VectorSubcoreMesh(core_axis_name="c", subcore_axis_name="s")
    out = jax.ShapeDtypeStruct((BATCH, SEQ), jnp.int32)

    @functools.partial(
        pl.kernel,
        mesh=mesh,
        out_type=[out, out],
        scratch_types=[
            pltpu.VMEM((CHUNK_ROWS, SEQ), jnp.int32),
            pltpu.VMEM((CHUNK_ROWS, SEQ), jnp.int32),
            pltpu.VMEM((CHUNK_ROWS, SEQ), jnp.int32),
        ],
        compiler_params=pltpu.CompilerParams(needs_layout_passes=False),
    )
    def run(x_hbm, p_hbm, r_hbm, in_v, p_v, r_v):
        wid = lax.axis_index("s") * 2 + lax.axis_index("c")
        lanes = lax.iota(jnp.int32, 16)

        def chunk_body(cidx, _):
            base = wid * ROWS_PER_WORKER + cidx * CHUNK_ROWS
            pltpu.sync_copy(x_hbm.at[pl.ds(base, CHUNK_ROWS)], in_v)

            def row_body(rr, _):
                rowv = jnp.full((16,), rr, jnp.int32)

                def vec_body(i, _):
                    jv = jnp.full((16,), i * 16, jnp.int32) + lanes
                    _emit(in_v, p_v, r_v, rowv, jv)
                    return 0

                return lax.fori_loop(0, VECS_PER_ROW, vec_body, 0)

            lax.fori_loop(0, CHUNK_ROWS, row_body, 0)
            rowv = lanes >> 1
            jv = jnp.full((16,), 2048, jnp.int32) + (lanes & 1)
            _emit(in_v, p_v, r_v, rowv, jv)

            pltpu.sync_copy(p_v, p_hbm.at[pl.ds(base, CHUNK_ROWS)])
            pltpu.sync_copy(r_v, r_hbm.at[pl.ds(base, CHUNK_ROWS)])
            return 0

        lax.fori_loop(0, CHUNKS, chunk_body, 0)

    return run


_RUN = _make_kernel()


def kernel(input_id_sequence):
    p, r = _RUN(input_id_sequence)
    return (p, r)


# speedup vs baseline: 1.3372x; 1.3372x over previous
import functools

import jax
import jax.numpy as jnp
from jax import lax
from jax.experimental import pallas as pl
from jax.experimental.pallas import tpu as pltpu
from jax.experimental.pallas import tpu_sc as plsc

BATCH = 4096
SEQ = 2050
NUM_WORKERS = 32
ROWS_PER_WORKER = BATCH // NUM_WORKERS
CHUNK_ROWS = 8
CHUNKS = ROWS_PER_WORKER // CHUNK_ROWS
VECS_PER_ROW = 2048 // 16


def _emit_tail(in_v, p_v, r_v, rowv, jv):
    w = plsc.load_gather(in_v, [rowv, jv])
    m = lax.rem(jv - 1, jnp.full((16,), 4, jnp.int32))
    tok = plsc.load_gather(in_v, [rowv, jv - m])
    inb = (jv >= 1) & (jv <= 2048)
    pos = jnp.where(inb & (w != 0) & (w != 1), m + 1, 0)
    rbt = jnp.where(inb & (m != 0) & (tok != 1), tok, 0)
    plsc.store_scatter(p_v, [rowv, jv], pos)
    plsc.store_scatter(r_v, [rowv, jv], rbt)


def _make_kernel():
    mesh = plsc.VectorSubcoreMesh(core_axis_name="c", subcore_axis_name="s")
    out = jax.ShapeDtypeStruct((BATCH, SEQ), jnp.int32)

    @functools.partial(
        pl.kernel,
        mesh=mesh,
        out_type=[out, out],
        scratch_types=[
            pltpu.VMEM((CHUNK_ROWS, SEQ), jnp.int32),
            pltpu.VMEM((CHUNK_ROWS, SEQ), jnp.int32),
            pltpu.VMEM((CHUNK_ROWS, SEQ), jnp.int32),
        ],
        compiler_params=pltpu.CompilerParams(needs_layout_passes=False),
    )
    def run(x_hbm, p_hbm, r_hbm, in_v, p_v, r_v):
        wid = lax.axis_index("s") * 2 + lax.axis_index("c")
        lanes = lax.iota(jnp.int32, 16)
        l0 = lanes == 0
        m = (lanes + 3) & 3
        posc = m + 1
        beh = m != 0
        idxc = (lanes - m) & 15
        c13 = jnp.full((16,), 13, jnp.int32)
        zero = jnp.zeros((16,), jnp.int32)

        def chunk_body(cidx, _):
            base = wid * ROWS_PER_WORKER + cidx * CHUNK_ROWS
            pltpu.sync_copy(x_hbm.at[pl.ds(base, CHUNK_ROWS)], in_v)

            def vec_body(i, carries):
                off = i * 16
                new = []
                for rr in range(CHUNK_ROWS):
                    x = in_v[rr, pl.ds(off, 16)]
                    g = jnp.take_along_axis(x, idxc, axis=0, mode="promise_in_bounds")
                    t = jnp.where(l0, carries[rr], g)
                    pos = jnp.where((x != 0) & (x != 1), posc, zero)
                    rbt = jnp.where(beh & (t != 1), t, zero)
                    p_v[rr, pl.ds(off, 16)] = pos
                    r_v[rr, pl.ds(off, 16)] = rbt
                    new.append(jnp.take_along_axis(x, c13, axis=0, mode="promise_in_bounds"))
                return tuple(new)

            lax.fori_loop(0, VECS_PER_ROW, vec_body,
                          tuple(zero for _ in range(CHUNK_ROWS)))

            for rr in range(CHUNK_ROWS):
                head = p_v[rr, pl.ds(0, 16)]
                p_v[rr, pl.ds(0, 16)] = jnp.where(l0, zero, head)

            rowv = lanes >> 1
            jv = jnp.full((16,), 2048, jnp.int32) + (lanes & 1)
            _emit_tail(in_v, p_v, r_v, rowv, jv)

            pltpu.sync_copy(p_v, p_hbm.at[pl.ds(base, CHUNK_ROWS)])
            pltpu.sync_copy(r_v, r_hbm.at[pl.ds(base, CHUNK_ROWS)])
            return 0

        lax.fori_loop(0, CHUNKS, chunk_body, 0)

    return run


_RUN = _make_kernel()


def kernel(input_id_sequence):
    p, r = _RUN(input_id_sequence)
    return (p, r)


# speedup vs baseline: 1.6860x; 1.2609x over previous
import functools

import jax
import jax.numpy as jnp
from jax import lax
from jax.experimental import pallas as pl
from jax.experimental.pallas import tpu as pltpu
from jax.experimental.pallas import tpu_sc as plsc

BATCH = 4096
SEQ = 2050
NUM_WORKERS = 32
ROWS_PER_WORKER = BATCH // NUM_WORKERS
CHUNK_ROWS = 8
CHUNKS = ROWS_PER_WORKER // CHUNK_ROWS
VECS_PER_ROW = 2048 // 16


def _emit_tail(in_v, p_v, r_v, rowv, jv):
    w = plsc.load_gather(in_v, [rowv, jv])
    m = lax.rem(jv - 1, jnp.full((16,), 4, jnp.int32))
    tok = plsc.load_gather(in_v, [rowv, jv - m])
    inb = (jv >= 1) & (jv <= 2048)
    pos = jnp.where(inb & (w != 0) & (w != 1), m + 1, 0)
    rbt = jnp.where(inb & (m != 0) & (tok != 1), tok, 0)
    plsc.store_scatter(p_v, [rowv, jv], pos)
    plsc.store_scatter(r_v, [rowv, jv], rbt)


def _make_kernel():
    mesh = plsc.VectorSubcoreMesh(core_axis_name="c", subcore_axis_name="s")
    out = jax.ShapeDtypeStruct((BATCH, SEQ), jnp.int32)
    buf = pltpu.VMEM((CHUNK_ROWS, SEQ), jnp.int32)

    @functools.partial(
        pl.kernel,
        mesh=mesh,
        out_type=[out, out],
        scratch_types=[buf] * 6 + [pltpu.SemaphoreType.DMA((2,)),
                                   pltpu.SemaphoreType.DMA((2,))],
        compiler_params=pltpu.CompilerParams(needs_layout_passes=False),
    )
    def run(x_hbm, p_hbm, r_hbm, in0, in1, p0, p1, r0, r1, si, so):
        ins, ps, rs = (in0, in1), (p0, p1), (r0, r1)
        wid = lax.axis_index("s") * 2 + lax.axis_index("c")
        row0 = wid * ROWS_PER_WORKER
        lanes = lax.iota(jnp.int32, 16)
        l0 = lanes == 0
        m = (lanes + 3) & 3
        posc = m + 1
        beh = m != 0
        idxc = (lanes - m) & 15
        c13 = jnp.full((16,), 13, jnp.int32)
        zero = jnp.zeros((16,), jnp.int32)

        def in_cp(c, b):
            return pltpu.make_async_copy(
                x_hbm.at[pl.ds(row0 + c * CHUNK_ROWS, CHUNK_ROWS)],
                ins[b], si.at[b])

        def outp_cp(c, b):
            return pltpu.make_async_copy(
                ps[b], p_hbm.at[pl.ds(row0 + c * CHUNK_ROWS, CHUNK_ROWS)],
                so.at[b])

        def outr_cp(c, b):
            return pltpu.make_async_copy(
                rs[b], r_hbm.at[pl.ds(row0 + c * CHUNK_ROWS, CHUNK_ROWS)],
                so.at[b])

        def compute(b):
            in_v, p_v, r_v = ins[b], ps[b], rs[b]

            def vec_body(i, carries):
                off = i * 16
                new = []
                for rr in range(CHUNK_ROWS):
                    x = in_v[rr, pl.ds(off, 16)]
                    g = jnp.take_along_axis(x, idxc, axis=0,
                                            mode="promise_in_bounds")
                    t = jnp.where(l0, carries[rr], g)
                    pos = jnp.where((x != 0) & (x != 1), posc, zero)
                    rbt = jnp.where(beh & (t != 1), t, zero)
                    p_v[rr, pl.ds(off, 16)] = pos
                    r_v[rr, pl.ds(off, 16)] = rbt
                    new.append(jnp.take_along_axis(x, c13, axis=0,
                                                   mode="promise_in_bounds"))
                return tuple(new)

            lax.fori_loop(0, VECS_PER_ROW, vec_body,
                          tuple(zero for _ in range(CHUNK_ROWS)))
            for rr in range(CHUNK_ROWS):
                head = p_v[rr, pl.ds(0, 16)]
                p_v[rr, pl.ds(0, 16)] = jnp.where(l0, zero, head)
            rowv = lanes >> 1
            jv = jnp.full((16,), 2048, jnp.int32) + (lanes & 1)
            _emit_tail(in_v, p_v, r_v, rowv, jv)

        in_cp(0, 0).start()

        @pl.loop(0, CHUNKS, step=2)
        def _(cs):
            for b in range(2):
                c = cs + b

                @pl.when(c + 1 < CHUNKS)
                def _():
                    in_cp(c + 1, b ^ 1).start()

                in_cp(c, b).wait()

                @pl.when(c >= 2)
                def _():
                    outp_cp(c - 2, b).wait()
                    outr_cp(c - 2, b).wait()

                compute(b)
                outp_cp(c, b).start()
                outr_cp(c, b).start()

        outp_cp(CHUNKS - 2, 0).wait()
        outr_cp(CHUNKS - 2, 0).wait()
        outp_cp(CHUNKS - 1, 1).wait()
        outr_cp(CHUNKS - 1, 1).wait()

    return run


_RUN = _make_kernel()


def kernel(input_id_sequence):
    p, r = _RUN(input_id_sequence)
    return (p, r)


# speedup vs baseline: 4.3796x; 2.5976x over previous
import functools

import jax
import jax.numpy as jnp
from jax import lax
from jax.experimental import pallas as pl
from jax.experimental.pallas import tpu as pltpu
from jax.experimental.pallas import tpu_sc as plsc

BATCH = 4096
SEQ = 2050
NUM_BANDS = 16
BAND_ROWS = 2048 // NUM_BANDS
HALF = BATCH // 2
CHUNK = 8
CHUNKS = BAND_ROWS // CHUNK
VECS = HALF // 16
POS_CONST = (4, 1, 2, 3, 4, 1, 2, 3)


def _make_kernel():
    mesh = plsc.VectorSubcoreMesh(core_axis_name="c", subcore_axis_name="s")
    out = jax.ShapeDtypeStruct((SEQ, BATCH), jnp.int32)
    buf = pltpu.VMEM((CHUNK, HALF), jnp.int32)

    @functools.partial(
        pl.kernel,
        mesh=mesh,
        out_type=[out, out],
        scratch_types=[buf] * 7 + [pltpu.SemaphoreType.DMA((3,)),
                                   pltpu.SemaphoreType.DMA((2,))],
        compiler_params=pltpu.CompilerParams(needs_layout_passes=False),
    )
    def run(x_hbm, p_hbm, r_hbm, in0, in1, in2, p0, p1, r0, r1, si, so):
        ins, ps, rs = (in0, in1, in2), (p0, p1), (r0, r1)
        wid = lax.axis_index("s") * 2 + lax.axis_index("c")
        band = wid >> 1
        col0 = (wid & 1) * HALF
        row_base = band * BAND_ROWS
        zero = jnp.zeros((16,), jnp.int32)

        def in_cp(t, b):
            return pltpu.make_async_copy(
                x_hbm.at[pl.ds(row_base + t * CHUNK, CHUNK),
                         pl.ds(col0, HALF)],
                ins[b], si.at[b])

        def outp_cp(t, b):
            return pltpu.make_async_copy(
                ps[b], p_hbm.at[pl.ds(row_base + t * CHUNK, CHUNK),
                                pl.ds(col0, HALF)], so.at[b])

        def outr_cp(t, b):
            return pltpu.make_async_copy(
                rs[b], r_hbm.at[pl.ds(row_base + t * CHUNK, CHUNK),
                                pl.ds(col0, HALF)], so.at[b])

        def compute(bi, bp, bo):
            in_v, prev_v = ins[bi], ins[bp]
            p_v, r_v = ps[bo], rs[bo]

            def vec_body(i, _):
                off = i * 16
                x1 = in_v[1, pl.ds(off, 16)]
                x5 = in_v[5, pl.ds(off, 16)]
                xp5 = prev_v[5, pl.ds(off, 16)]
                rbt_a = jnp.where(x1 >= 2, x1, zero)
                rbt_b = jnp.where(x5 >= 2, x5, zero)
                rbt_c = jnp.where(xp5 >= 2, xp5, zero)
                for rr in range(CHUNK):
                    x = in_v[rr, pl.ds(off, 16)]
                    p_v[rr, pl.ds(off, 16)] = jnp.where(
                        x >= 2, POS_CONST[rr], 0)
                r_v[0, pl.ds(off, 16)] = rbt_c
                r_v[1, pl.ds(off, 16)] = zero
                r_v[2, pl.ds(off, 16)] = rbt_a
                r_v[3, pl.ds(off, 16)] = rbt_a
                r_v[4, pl.ds(off, 16)] = rbt_a
                r_v[5, pl.ds(off, 16)] = zero
                r_v[6, pl.ds(off, 16)] = rbt_b
                r_v[7, pl.ds(off, 16)] = rbt_b
                return 0

            lax.fori_loop(0, VECS, vec_body, 0)

        prev_base = jnp.where(band == 0, 0, row_base - CHUNK)
        pltpu.async_copy(
            x_hbm.at[pl.ds(prev_base, CHUNK), pl.ds(col0, HALF)],
            ins[2], si.at[2]).wait()
        in_cp(0, 0).start()

        @pl.loop(0, 18, step=6)
        def _(ts):
            for b in range(6):
                t = ts + b
                bi, bo = b % 3, b % 2

                @pl.when(t < CHUNKS)
                def _():
                    @pl.when(t + 1 < CHUNKS)
                    def _():
                        in_cp(t + 1, (bi + 1) % 3).start()

                    in_cp(t, bi).wait()

                    @pl.when(t >= 2)
                    def _():
                        outp_cp(t - 2, bo).wait()
                        outr_cp(t - 2, bo).wait()

                    compute(bi, (bi + 2) % 3, bo)

                    @pl.when((band == 0) & (t == 0))
                    def _():
                        def z0(i, _):
                            ps[0][0, pl.ds(i * 16, 16)] = zero
                            rs[0][0, pl.ds(i * 16, 16)] = zero
                            return 0

                        lax.fori_loop(0, VECS, z0, 0)

                    outp_cp(t, bo).start()
                    outr_cp(t, bo).start()

        outp_cp(CHUNKS - 2, 0).wait()
        outr_cp(CHUNKS - 2, 0).wait()
        outp_cp(CHUNKS - 1, 1).wait()
        outr_cp(CHUNKS - 1, 1).wait()

        @pl.when(band == NUM_BANDS - 1)
        def _():
            pltpu.async_copy(
                x_hbm.at[pl.ds(2048, 2), pl.ds(col0, HALF)],
                ins[1].at[pl.ds(0, 2)], si.at[1]).wait()

            def tail_body(i, _):
                off = i * 16
                x = ins[1][0, pl.ds(off, 16)]
                tok = ins[0][5, pl.ds(off, 16)]
                ps[0][0, pl.ds(off, 16)] = jnp.where(x >= 2, 4, 0)
                rs[0][0, pl.ds(off, 16)] = jnp.where(tok >= 2, tok, zero)
                ps[0][1, pl.ds(off, 16)] = zero
                rs[0][1, pl.ds(off, 16)] = zero
                return 0

            lax.fori_loop(0, VECS, tail_body, 0)
            pltpu.sync_copy(ps[0].at[pl.ds(0, 2)],
                            p_hbm.at[pl.ds(2048, 2), pl.ds(col0, HALF)])
            pltpu.sync_copy(rs[0].at[pl.ds(0, 2)],
                            r_hbm.at[pl.ds(2048, 2), pl.ds(col0, HALF)])

    return run


_RUN = _make_kernel()


def kernel(input_id_sequence):
    pt, rt = _RUN(input_id_sequence.T)
    return (pt.T, rt.T)


# speedup vs baseline: 4.4871x; 1.0245x over previous
import functools

import jax
import jax.numpy as jnp
from jax import lax
from jax.experimental import pallas as pl
from jax.experimental.pallas import tpu as pltpu
from jax.experimental.pallas import tpu_sc as plsc

BATCH = 4096
SEQ = 2050
NUM_BANDS = 16
BAND_ROWS = 2048 // NUM_BANDS
HALF = BATCH // 2
CHUNK = 8
CHUNKS = BAND_ROWS // CHUNK
VECS = HALF // 16
POS_CONST = (4, 1, 2, 3, 4, 1, 2, 3)


def _make_kernel():
    mesh = plsc.VectorSubcoreMesh(core_axis_name="c", subcore_axis_name="s")
    out = jax.ShapeDtypeStruct((SEQ, BATCH), jnp.int32)
    buf = pltpu.VMEM((CHUNK, HALF), jnp.int32)

    @functools.partial(
        pl.kernel,
        mesh=mesh,
        out_type=[out, out],
        scratch_types=[buf] * 7 + [pltpu.SemaphoreType.DMA((3,)),
                                   pltpu.SemaphoreType.DMA((2,))],
        compiler_params=pltpu.CompilerParams(needs_layout_passes=False),
    )
    def run(x_hbm, p_hbm, r_hbm, in0, in1, in2, p0, p1, r0, r1, si, so):
        ins, ps, rs = (in0, in1, in2), (p0, p1), (r0, r1)
        wid = lax.axis_index("s") * 2 + lax.axis_index("c")
        band = wid >> 1
        col0 = (wid & 1) * HALF
        row_base = band * BAND_ROWS
        zero = jnp.zeros((16,), jnp.int32)

        def in_cp(t, b):
            return pltpu.make_async_copy(
                x_hbm.at[pl.ds(row_base + t * CHUNK, CHUNK),
                         pl.ds(col0, HALF)],
                ins[b], si.at[b])

        def outp_cp(t, b):
            return pltpu.make_async_copy(
                ps[b], p_hbm.at[pl.ds(row_base + t * CHUNK, CHUNK),
                                pl.ds(col0, HALF)], so.at[b])

        def outr_cp(t, b):
            return pltpu.make_async_copy(
                rs[b], r_hbm.at[pl.ds(row_base + t * CHUNK, CHUNK),
                                pl.ds(col0, HALF)], so.at[b])

        def compute(bi, bp, bo):
            in_v, prev_v = ins[bi], ins[bp]
            p_v, r_v = ps[bo], rs[bo]

            def vec_body(i, _):
                off = i * 16
                xs = [in_v[rr, pl.ds(off, 16)] for rr in range(CHUNK)]
                xp5 = prev_v[5, pl.ds(off, 16)]
                rbt_a = jnp.where(xs[1] >= 2, xs[1], zero)
                rbt_b = jnp.where(xs[5] >= 2, xs[5], zero)
                rbt_c = jnp.where(xp5 >= 2, xp5, zero)
                for rr in range(CHUNK):
                    p_v[rr, pl.ds(off, 16)] = jnp.where(
                        xs[rr] >= 2, POS_CONST[rr], 0)
                r_v[0, pl.ds(off, 16)] = rbt_c
                r_v[1, pl.ds(off, 16)] = zero
                r_v[2, pl.ds(off, 16)] = rbt_a
                r_v[3, pl.ds(off, 16)] = rbt_a
                r_v[4, pl.ds(off, 16)] = rbt_a
                r_v[5, pl.ds(off, 16)] = zero
                r_v[6, pl.ds(off, 16)] = rbt_b
                r_v[7, pl.ds(off, 16)] = rbt_b
                return 0

            lax.fori_loop(0, VECS, vec_body, 0)

        prev_base = jnp.where(band == 0, 0, row_base - CHUNK)
        pltpu.async_copy(
            x_hbm.at[pl.ds(prev_base, CHUNK), pl.ds(col0, HALF)],
            ins[2], si.at[2]).wait()
        in_cp(0, 0).start()

        @pl.loop(0, 18, step=6)
        def _(ts):
            for b in range(6):
                t = ts + b
                bi, bo = b % 3, b % 2

                @pl.when(t < CHUNKS)
                def _():
                    @pl.when(t + 1 < CHUNKS)
                    def _():
                        in_cp(t + 1, (bi + 1) % 3).start()

                    in_cp(t, bi).wait()

                    @pl.when(t >= 2)
                    def _():
                        outp_cp(t - 2, bo).wait()
                        outr_cp(t - 2, bo).wait()

                    compute(bi, (bi + 2) % 3, bo)

                    @pl.when((band == 0) & (t == 0))
                    def _():
                        def z0(i, _):
                            ps[0][0, pl.ds(i * 16, 16)] = zero
                            rs[0][0, pl.ds(i * 16, 16)] = zero
                            return 0

                        lax.fori_loop(0, VECS, z0, 0)

                    outp_cp(t, bo).start()
                    outr_cp(t, bo).start()

        outp_cp(CHUNKS - 2, 0).wait()
        outr_cp(CHUNKS - 2, 0).wait()
        outp_cp(CHUNKS - 1, 1).wait()
        outr_cp(CHUNKS - 1, 1).wait()

        @pl.when(band == NUM_BANDS - 1)
        def _():
            pltpu.async_copy(
                x_hbm.at[pl.ds(2048, 2), pl.ds(col0, HALF)],
                ins[1].at[pl.ds(0, 2)], si.at[1]).wait()

            def tail_body(i, _):
                off = i * 16
                x = ins[1][0, pl.ds(off, 16)]
                tok = ins[0][5, pl.ds(off, 16)]
                ps[0][0, pl.ds(off, 16)] = jnp.where(x >= 2, 4, 0)
                rs[0][0, pl.ds(off, 16)] = jnp.where(tok >= 2, tok, zero)
                ps[0][1, pl.ds(off, 16)] = zero
                rs[0][1, pl.ds(off, 16)] = zero
                return 0

            lax.fori_loop(0, VECS, tail_body, 0)
            pltpu.sync_copy(ps[0].at[pl.ds(0, 2)],
                            p_hbm.at[pl.ds(2048, 2), pl.ds(col0, HALF)])
            pltpu.sync_copy(rs[0].at[pl.ds(0, 2)],
                            r_hbm.at[pl.ds(2048, 2), pl.ds(col0, HALF)])

    return run


_RUN = _make_kernel()


def kernel(input_id_sequence):
    pt, rt = _RUN(input_id_sequence.T)
    return (pt.T, rt.T)
